# baseline (speedup 1.0000x reference)
import jax
import jax.numpy as jnp
from jax import lax
from jax.experimental import pallas as pl
from jax.experimental.pallas import tpu as pltpu
from jax.experimental.pallas import tpu_sc as plsc

B_ROWS = 500000
D = 128
OUT_D = 4 * D
C = 80
NC = 2
NS = 16
NW = NC * NS
N_CHUNKS = B_ROWS // C
MAX_J = (N_CHUNKS + NW - 1) // NW
N_TRIPLES = (MAX_J + 2) // 3
NBUF = 3
LANES = 16
P_DC = 2000
N_PIECES = B_ROWS // P_DC
MAX_Q = (N_PIECES + NW - 1) // NW


def _body(z_src, z_dst, idx, t_enc, table, out,
          idx_v0, idx_v1, idx_v2, zs_v0, zs_v1, zs_v2, g_v0, g_v1, g_v2,
          sem_i0, sem_i1, sem_i2, sem_l0, sem_l1, sem_l2,
          sem_g0, sem_g1, sem_g2, sem_w0, sem_w1, sem_w2, sem_dc):
    wid = lax.axis_index("s") * NC + lax.axis_index("c")
    idx_v = (idx_v0, idx_v1, idx_v2)
    zs_v = (zs_v0, zs_v1, zs_v2)
    g_v = (g_v0, g_v1, g_v2)
    sem_i = (sem_i0, sem_i1, sem_i2)
    sem_l = (sem_l0, sem_l1, sem_l2)
    sem_g = (sem_g0, sem_g1, sem_g2)
    sem_w = (sem_w0, sem_w1, sem_w2)

    def rows_of(cid):
        return pl.ds(cid * C, C)

    def zs_desc(s, rows):
        return pltpu.make_async_copy(z_src.at[rows], zs_v[s], sem_l[s])

    def write_descs(s, rows):
        return (
            pltpu.make_async_copy(zs_v[s], out.at[rows, pl.ds(0, D)], sem_w[s]),
            pltpu.make_async_copy(g_v[s], out.at[rows, pl.ds(2 * D, D)], sem_w[s]),
        )

    def dc_descs(piece):
        rows = pl.ds(piece * P_DC, P_DC)
        return (
            pltpu.make_async_copy(z_dst.at[rows], out.at[rows, pl.ds(D, D)], sem_dc),
            pltpu.make_async_copy(t_enc.at[rows], out.at[rows, pl.ds(3 * D, D)], sem_dc),
        )

    def issue_loads(s, cid):
        rows = rows_of(cid)
        pltpu.make_async_copy(idx.at[rows], idx_v[s], sem_i[s]).start()
        zs_desc(s, rows).start()

    issue_loads(0, wid)
    issue_loads(1, wid + NW)
    pltpu.make_async_copy(idx.at[rows_of(wid)], idx_v[0], sem_i[0]).wait()
    pltpu.make_async_copy(table.at[idx_v[0]], g_v[0], sem_g[0]).start()

    def triple_body(t, carry):
        for u in range(NBUF):
            cid = wid + (NBUF * t + u) * NW
            s = u
            s1 = (u + 1) % NBUF
            s2 = (u + 2) % NBUF

            if u == 0:
                for q in range(MAX_Q):
                    @pl.when((t == q * (N_TRIPLES // MAX_Q))
                             & (wid + q * NW < N_PIECES))
                    def _():
                        for d in dc_descs(wid + q * NW):
                            d.start()

            @pl.when(cid < N_CHUNKS)
            def _():
                rows = rows_of(cid)
                zs_desc(s, rows).wait()
                write_descs(s, rows)[0].start()

            @pl.when(cid + NW < N_CHUNKS)
            def _():
                rows1 = rows_of(cid + NW)
                pltpu.make_async_copy(idx.at[rows1], idx_v[s1], sem_i[s1]).wait()
                pltpu.make_async_copy(table.at[idx_v[s1]], g_v[s1], sem_g[s1]).start()

            @pl.when(cid < N_CHUNKS)
            def _():
                rows = rows_of(cid)
                pltpu.make_async_copy(table.at[idx_v[s]], g_v[s], sem_g[s]).wait()

                def row_body(r, c2):
                    for kk in range(D // LANES):
                        sl = pl.ds(kk * LANES, LANES)
                        plsc.addupdate(g_v[s].at[r, sl], zs_v[s][r, sl])
                    return c2

                lax.fori_loop(0, C, row_body, 0, unroll=4)
                write_descs(s, rows)[1].start()

            has_prev = (cid + 2 * NW < N_CHUNKS)
            if u == 0:
                has_prev = has_prev & (t >= 1)

            @pl.when(has_prev)
            def _():
                for d in write_descs(s2, rows_of(cid - NW)):
                    d.wait()

            @pl.when(cid + 2 * NW < N_CHUNKS)
            def _():
                issue_loads(s2, cid + 2 * NW)

        return carry

    lax.fori_loop(0, N_TRIPLES, triple_body, 0)

    for s in range(NBUF):
        for d in write_descs(s, rows_of(wid)):
            d.wait()
    for q in range(MAX_Q):
        @pl.when(wid + q * NW < N_PIECES)
        def _():
            for d in dc_descs(wid + q * NW):
                d.wait()


def kernel(z_src, z_dst, raw_msg, t_enc, emb_table):
    mesh = plsc.VectorSubcoreMesh(core_axis_name="c", subcore_axis_name="s")
    run = pl.kernel(
        _body,
        out_type=jax.ShapeDtypeStruct((B_ROWS, OUT_D), jnp.float32),
        mesh=mesh,
        scratch_types=(
            [pltpu.VMEM((C,), jnp.int32)] * 3
            + [pltpu.VMEM((C, D), jnp.float32)] * 6
            + [pltpu.SemaphoreType.DMA] * 13
        ),
    )
    return run(z_src, z_dst, raw_msg.astype(jnp.int32), t_enc, emb_table)


# speedup vs baseline: 21.2012x; 21.2012x over previous
import jax
import jax.numpy as jnp
from jax import lax
from jax.experimental import pallas as pl
from jax.experimental.pallas import tpu as pltpu
from jax.experimental.pallas import tpu_sc as plsc

B_ROWS = 500000
D = 128
OUT_D = 4 * D
C = 80
NC = 2
NS = 16
NW = NC * NS
N_CHUNKS = B_ROWS // C
MAX_J = (N_CHUNKS + NW - 1) // NW
N_TRIPLES = (MAX_J + 2) // 3
NBUF = 3
LANES = 16


def _body(z_src, z_dst, idx, t_enc, table, out,
          idx_v0, idx_v1, idx_v2, a_v0, a_v1, a_v2,
          sem_i0, sem_i1, sem_i2, sem_l0, sem_l1, sem_l2,
          sem_g0, sem_g1, sem_g2, sem_w0, sem_w1, sem_w2):
    wid = lax.axis_index("s") * NC + lax.axis_index("c")
    idx_v = (idx_v0, idx_v1, idx_v2)
    a_v = (a_v0, a_v1, a_v2)
    sem_i = (sem_i0, sem_i1, sem_i2)
    sem_l = (sem_l0, sem_l1, sem_l2)
    sem_g = (sem_g0, sem_g1, sem_g2)
    sem_w = (sem_w0, sem_w1, sem_w2)

    def rows_of(cid):
        return pl.ds(cid * C, C)

    def load_descs(s, rows):
        return (
            pltpu.make_async_copy(z_src.at[rows], a_v[s].at[:, pl.ds(0, D)], sem_l[s]),
            pltpu.make_async_copy(z_dst.at[rows], a_v[s].at[:, pl.ds(D, D)], sem_l[s]),
            pltpu.make_async_copy(t_enc.at[rows], a_v[s].at[:, pl.ds(3 * D, D)], sem_l[s]),
        )

    def write_desc(s, rows):
        return pltpu.make_async_copy(a_v[s], out.at[rows], sem_w[s])

    def gather_desc(s):
        return pltpu.make_async_copy(
            table.at[idx_v[s]], a_v[s].at[:, pl.ds(2 * D, D)], sem_g[s])

    def issue_loads(s, cid):
        rows = rows_of(cid)
        pltpu.make_async_copy(idx.at[rows], idx_v[s], sem_i[s]).start()
        for d in load_descs(s, rows):
            d.start()

    issue_loads(0, wid)
    issue_loads(1, wid + NW)
    pltpu.make_async_copy(idx.at[rows_of(wid)], idx_v[0], sem_i[0]).wait()
    gather_desc(0).start()

    def triple_body(t, carry):
        for u in range(NBUF):
            cid = wid + (NBUF * t + u) * NW
            s = u
            s1 = (u + 1) % NBUF
            s2 = (u + 2) % NBUF

            @pl.when(cid < N_CHUNKS)
            def _():
                for d in load_descs(s, rows_of(cid)):
                    d.wait()

            @pl.when(cid + NW < N_CHUNKS)
            def _():
                rows1 = rows_of(cid + NW)
                pltpu.make_async_copy(idx.at[rows1], idx_v[s1], sem_i[s1]).wait()
                gather_desc(s1).start()

            @pl.when(cid < N_CHUNKS)
            def _():
                gather_desc(s).wait()

                def row_body(r, c2):
                    for kk in range(D // LANES):
                        plsc.addupdate(
                            a_v[s].at[r, pl.ds(2 * D + kk * LANES, LANES)],
                            a_v[s][r, pl.ds(kk * LANES, LANES)])
                    return c2

                lax.fori_loop(0, C, row_body, 0, unroll=4)
                write_desc(s, rows_of(cid)).start()

            has_prev = (cid + 2 * NW < N_CHUNKS)
            if u == 0:
                has_prev = has_prev & (t >= 1)

            @pl.when(has_prev)
            def _():
                write_desc(s2, rows_of(cid - NW)).wait()

            @pl.when(cid + 2 * NW < N_CHUNKS)
            def _():
                issue_loads(s2, cid + 2 * NW)

        return carry

    lax.fori_loop(0, N_TRIPLES, triple_body, 0)

    for s in range(NBUF):
        write_desc(s, rows_of(wid)).wait()


def kernel(z_src, z_dst, raw_msg, t_enc, emb_table):
    mesh = plsc.VectorSubcoreMesh(core_axis_name="c", subcore_axis_name="s")
    run = pl.kernel(
        _body,
        out_type=jax.ShapeDtypeStruct((B_ROWS, OUT_D), jnp.float32),
        mesh=mesh,
        scratch_types=(
            [pltpu.VMEM((C,), jnp.int32)] * 3
            + [pltpu.VMEM((C, OUT_D), jnp.float32)] * 3
            + [pltpu.SemaphoreType.DMA] * 12
        ),
    )
    return run(z_src, z_dst, raw_msg.astype(jnp.int32), t_enc, emb_table)


# speedup vs baseline: 21.4533x; 1.0119x over previous
import jax
import jax.numpy as jnp
from jax import lax
from jax.experimental import pallas as pl
from jax.experimental.pallas import tpu as pltpu
from jax.experimental.pallas import tpu_sc as plsc

B_ROWS = 500000
D = 128
OUT_D = 4 * D
C = 80
NC = 2
NS = 16
NW = NC * NS
N_CHUNKS = B_ROWS // C
MAX_J = (N_CHUNKS + NW - 1) // NW
N_TRIPLES = (MAX_J + 2) // 3
NBUF = 3
LANES = 16


def _body(z_src, z_dst, idx, t_enc, table, out,
          idx_v0, idx_v1, idx_v2, a_v0, a_v1, a_v2,
          sem_i0, sem_i1, sem_i2, sem_l0, sem_l1, sem_l2,
          sem_g0, sem_g1, sem_g2, sem_w0, sem_w1, sem_w2):
    wid = lax.axis_index("s") * NC + lax.axis_index("c")
    idx_v = (idx_v0, idx_v1, idx_v2)
    a_v = (a_v0, a_v1, a_v2)
    sem_i = (sem_i0, sem_i1, sem_i2)
    sem_l = (sem_l0, sem_l1, sem_l2)
    sem_g = (sem_g0, sem_g1, sem_g2)
    sem_w = (sem_w0, sem_w1, sem_w2)

    def rows_of(cid):
        return pl.ds(cid * C, C)

    def load_descs(s, rows):
        return (
            pltpu.make_async_copy(z_src.at[rows], a_v[s].at[:, pl.ds(0, D)], sem_l[s]),
            pltpu.make_async_copy(z_dst.at[rows], a_v[s].at[:, pl.ds(D, D)], sem_l[s]),
            pltpu.make_async_copy(t_enc.at[rows], a_v[s].at[:, pl.ds(3 * D, D)], sem_l[s]),
        )

    def write_desc(s, rows):
        return pltpu.make_async_copy(a_v[s], out.at[rows], sem_w[s])

    def gather_desc(s):
        return pltpu.make_async_copy(
            table.at[idx_v[s]], a_v[s].at[:, pl.ds(2 * D, D)], sem_g[s])

    def issue_loads(s, cid):
        rows = rows_of(cid)
        pltpu.make_async_copy(idx.at[rows], idx_v[s], sem_i[s]).start()
        for d in load_descs(s, rows):
            d.start()

    issue_loads(0, wid)
    issue_loads(1, wid + NW)
    pltpu.make_async_copy(idx.at[rows_of(wid)], idx_v[0], sem_i[0]).wait()
    gather_desc(0).start()

    def triple_body(t, carry):
        for u in range(NBUF):
            cid = wid + (NBUF * t + u) * NW
            s = u
            s1 = (u + 1) % NBUF
            s2 = (u + 2) % NBUF

            @pl.when(cid < N_CHUNKS)
            def _():
                for d in load_descs(s, rows_of(cid)):
                    d.wait()

            @pl.when(cid + NW < N_CHUNKS)
            def _():
                rows1 = rows_of(cid + NW)
                pltpu.make_async_copy(idx.at[rows1], idx_v[s1], sem_i[s1]).wait()
                gather_desc(s1).start()

            @pl.when(cid < N_CHUNKS)
            def _():
                gather_desc(s).wait()

                @plsc.parallel_loop(0, C, unroll=4)
                def _(r):
                    for kk in range(D // LANES):
                        plsc.addupdate(
                            a_v[s].at[r, pl.ds(2 * D + kk * LANES, LANES)],
                            a_v[s][r, pl.ds(kk * LANES, LANES)])

                write_desc(s, rows_of(cid)).start()

            has_prev = (cid + 2 * NW < N_CHUNKS)
            if u == 0:
                has_prev = has_prev & (t >= 1)

            @pl.when(has_prev)
            def _():
                write_desc(s2, rows_of(cid - NW)).wait()

            @pl.when(cid + 2 * NW < N_CHUNKS)
            def _():
                issue_loads(s2, cid + 2 * NW)

        return carry

    lax.fori_loop(0, N_TRIPLES, triple_body, 0)

    for s in range(NBUF):
        write_desc(s, rows_of(wid)).wait()


def kernel(z_src, z_dst, raw_msg, t_enc, emb_table):
    mesh = plsc.VectorSubcoreMesh(core_axis_name="c", subcore_axis_name="s")
    run = pl.kernel(
        _body,
        out_type=jax.ShapeDtypeStruct((B_ROWS, OUT_D), jnp.float32),
        mesh=mesh,
        scratch_types=(
            [pltpu.VMEM((C,), jnp.int32)] * 3
            + [pltpu.VMEM((C, OUT_D), jnp.float32)] * 3
            + [pltpu.SemaphoreType.DMA] * 12
        ),
    )
    return run(z_src, z_dst, raw_msg.astype(jnp.int32), t_enc, emb_table)
